# Initial kernel scaffold; baseline (speedup 1.0000x reference)
import jax
import jax.numpy as jnp
from jax.experimental import pallas as pl


def kernel(x, edge_index, edge_attr, params):
    raise NotImplementedError("write your pallas kernel here")


# speedup vs baseline: 6.7110x; 6.7110x over previous
import functools

import jax
import jax.numpy as jnp
from jax import lax
from jax.experimental import pallas as pl
from jax.experimental.pallas import tpu as pltpu
from jax.experimental.pallas import tpu_sc as plsc

N = 10000
E = 320000
F = 128
NW = 32
EPW = E // NW
C = 128
NCH = -(-EPW // C)
EPWP = NCH * C
NPAD = 10240
RPT = NPAD // 16

_MESH = plsc.VectorSubcoreMesh(core_axis_name="c", subcore_axis_name="s")
_SC_PARAMS = pltpu.CompilerParams(needs_layout_passes=False)


@functools.partial(
    pl.kernel,
    out_type=jax.ShapeDtypeStruct((2, NPAD), jnp.float32),
    mesh=_MESH,
    compiler_params=_SC_PARAMS,
    scratch_types=[
        pltpu.VMEM((EPWP,), jnp.int32),
        pltpu.VMEM((NPAD,), jnp.float32),
        pltpu.VMEM((16, RPT), jnp.float32),
        pltpu.VMEM((RPT,), jnp.float32),
        pltpu.VMEM_SHARED((16, NPAD), jnp.float32),
    ],
)
def _deg_kernel(coli, zeros1d, out, colv, hist, blk, red, hist_sh):
    c = lax.axis_index("c")
    s = lax.axis_index("s")
    wid = s * 2 + c
    pltpu.sync_copy(coli.at[wid], colv)
    pltpu.sync_copy(zeros1d, hist)
    ones = jnp.ones((16,), jnp.float32)

    def body(i, carry):
        idx = colv[pl.ds(i * 16, 16)]
        plsc.addupdate_scatter(hist, [idx], ones)
        return carry

    lax.fori_loop(0, EPWP // 16, body, 0)
    pltpu.sync_copy(hist, hist_sh.at[s])
    plsc.subcore_barrier()
    for r in range(16):
        pltpu.sync_copy(hist_sh.at[r, pl.ds(s * RPT, RPT)], blk.at[r])
    for v in range(RPT // 16):
        a = blk[0, pl.ds(v * 16, 16)]
        for r in range(1, 16):
            a = a + blk[r, pl.ds(v * 16, 16)]
        red[pl.ds(v * 16, 16)] = a
    pltpu.sync_copy(red, out.at[c, pl.ds(s * RPT, RPT)])


@functools.partial(
    pl.kernel,
    out_type=jax.ShapeDtypeStruct((2, NPAD), jnp.float32),
    mesh=_MESH,
    compiler_params=_SC_PARAMS,
    scratch_types=[
        pltpu.VMEM((EPWP,), jnp.int32),
        pltpu.VMEM((EPWP,), jnp.int32),
        pltpu.VMEM((EPWP,), jnp.float32),
        pltpu.VMEM((NPAD,), jnp.float32),
        pltpu.VMEM((NPAD,), jnp.float32),
        pltpu.VMEM((16, RPT), jnp.float32),
        pltpu.VMEM((RPT,), jnp.float32),
        pltpu.VMEM_SHARED((16, NPAD), jnp.float32),
    ],
)
def _s_kernel(rowi, coli, eai, dis_pad, zeros1d, out,
              rowv, colv, eav, disv, hist, blk, red, hist_sh):
    c = lax.axis_index("c")
    s = lax.axis_index("s")
    wid = s * 2 + c
    pltpu.sync_copy(rowi.at[wid], rowv)
    pltpu.sync_copy(coli.at[wid], colv)
    pltpu.sync_copy(eai.at[wid], eav)
    pltpu.sync_copy(dis_pad, disv)
    pltpu.sync_copy(zeros1d, hist)

    def body(i, carry):
        r16 = rowv[pl.ds(i * 16, 16)]
        c16 = colv[pl.ds(i * 16, 16)]
        e16 = eav[pl.ds(i * 16, 16)]
        dr = plsc.load_gather(disv, [r16])
        plsc.addupdate_scatter(hist, [c16], dr * e16)
        return carry

    lax.fori_loop(0, EPWP // 16, body, 0)
    pltpu.sync_copy(hist, hist_sh.at[s])
    plsc.subcore_barrier()
    for r in range(16):
        pltpu.sync_copy(hist_sh.at[r, pl.ds(s * RPT, RPT)], blk.at[r])
    for v in range(RPT // 16):
        a = blk[0, pl.ds(v * 16, 16)]
        for r in range(1, 16):
            a = a + blk[r, pl.ds(v * 16, 16)]
        red[pl.ds(v * 16, 16)] = a
    pltpu.sync_copy(red, out.at[c, pl.ds(s * RPT, RPT)])


NSPLIT = NPAD // 2
DUMMY = NSPLIT
RPS = NSPLIT // 16


@functools.partial(
    pl.kernel,
    out_type=jax.ShapeDtypeStruct((2, NPAD, F), jnp.float32),
    mesh=_MESH,
    compiler_params=_SC_PARAMS,
    scratch_types=[
        pltpu.VMEM((NCH, C), jnp.int32),
        pltpu.VMEM((NCH, C), jnp.int32),
        pltpu.VMEM((NCH, C), jnp.int32),
        pltpu.VMEM((C, F), jnp.float32),
        pltpu.VMEM_SHARED((NSPLIT + C, F), jnp.float32),
        pltpu.SemaphoreType.DMA,
        pltpu.SemaphoreType.DMA,
    ],
)
def _pass_kernel(ytab, rowi, coli, zeros2d, out,
                 rowv, colv, colp, buf0, acc, g0, s0):
    c = lax.axis_index("c")
    s = lax.axis_index("s")
    wid = s * 2 + c
    pltpu.sync_copy(rowi.at[wid], rowv)
    pltpu.sync_copy(coli.at[wid], colv)
    for p in range(2):
        base = p * NSPLIT
        def cbody(g, carry, base=base):
            j = g // 8
            k = (g % 8) * 16
            c16 = colv[j, pl.ds(k, 16)]
            ok = (c16 >= base) & (c16 < base + NSPLIT)
            colp[j, pl.ds(k, 16)] = jnp.where(ok, c16 - base, DUMMY)
            return carry

        lax.fori_loop(0, NCH * 8, cbody, 0)
        pltpu.sync_copy(zeros2d, buf0)
        pltpu.sync_copy(buf0, acc.at[pl.ds(s * RPS, C)])
        pltpu.sync_copy(buf0, acc.at[pl.ds(s * RPS + C, C)])
        pltpu.sync_copy(buf0.at[pl.ds(0, RPS - 2 * C)], acc.at[pl.ds(s * RPS + 2 * C, RPS - 2 * C)])

        @pl.when(s == 0)
        def _():
            pltpu.sync_copy(buf0, acc.at[pl.ds(NSPLIT, C)])

        plsc.subcore_barrier()

        def body(j, carry):
            pltpu.async_copy(ytab.at[rowv.at[j]], buf0, g0).wait()
            pltpu.async_copy(buf0, acc.at[colp.at[j]], s0, add=True).wait()
            return carry

        lax.fori_loop(0, NCH, body, 0)
        plsc.subcore_barrier()
        for r0, cnt in ((0, C), (C, C), (2 * C, RPS - 2 * C)):
            pltpu.sync_copy(acc.at[pl.ds(s * RPS + r0, cnt)], buf0.at[pl.ds(0, cnt)])
            pltpu.sync_copy(buf0.at[pl.ds(0, cnt)], out.at[c, pl.ds(base + s * RPS + r0, cnt)])


def _bn(z, g, b):
    mu = jnp.mean(z, axis=0)
    var = jnp.mean((z - mu) ** 2, axis=0)
    return g * (z - mu) * lax.rsqrt(var + 1e-5) + b


def _dis_body(degp, dis_pad):
    deg = degp[0] + degp[1] + (1.0 + 1e-6)
    dis_pad[...] = lax.rsqrt(deg)


def _prep_body(x, dis, nw, new_w, ytab):
    w = jnp.dot(nw[...], new_w[:F], preferred_element_type=jnp.float32)
    y = jnp.dot(x[...], w, preferred_element_type=jnp.float32)
    ytab[...] = dis[...][:, None] * y


def _post1_body(accp, sp, dis, ytab1, x, ew, new_w, b1, bn_g, bn_b,
                l1_w, l1_b, pih_w, pih_b, c2_nw, c2_new, ytab2, pih):
    s_ = sp[0] + sp[1]
    v1 = jnp.dot(ew[...], new_w[F:], preferred_element_type=jnp.float32)
    agg = accp[0] + accp[1] + s_[:, None] * v1 + ytab1[...]
    out1 = dis[...][:, None] * agg + b1[...]
    x1 = jax.nn.relu(_bn(out1, bn_g[...], bn_b[...])) + x[...]
    z2 = jnp.dot(x1, l1_w[...], preferred_element_type=jnp.float32) + l1_b[...]
    w2 = jnp.dot(c2_nw[...], c2_new[:F], preferred_element_type=jnp.float32)
    ytab2[...] = dis[...][:, None] * jnp.dot(z2, w2, preferred_element_type=jnp.float32)
    pih[...] = jnp.dot(x1, pih_w[...], preferred_element_type=jnp.float32) + pih_b[...]


def _post2_body(accp, sp, dis, ytab2, pihr, ew, new_w, b2, bn_g, bn_b,
                l2_w, l2_b, pho_w, pho_b, m_nw, m_new, s_nw, s_new,
                ytab3m, ytab3s, pho):
    s_ = sp[0] + sp[1]
    v2 = jnp.dot(ew[...], new_w[F:], preferred_element_type=jnp.float32)
    agg = accp[0] + accp[1] + s_[:, None] * v2 + ytab2[...]
    out2 = dis[...][:, None] * agg + b2[...]
    x2 = jax.nn.relu(_bn(out2, bn_g[...], bn_b[...])) + pihr[...]
    x3 = jnp.dot(x2, l2_w[...], preferred_element_type=jnp.float32) + l2_b[...]
    wm = jnp.dot(m_nw[...], m_new[:64], preferred_element_type=jnp.float32)
    ws = jnp.dot(s_nw[...], s_new[:64], preferred_element_type=jnp.float32)
    ytab3m[...] = dis[...][:, None] * jnp.dot(x3, wm, preferred_element_type=jnp.float32)
    ytab3s[...] = dis[...][:, None] * jnp.dot(x3, ws, preferred_element_type=jnp.float32)
    pho[...] = jnp.dot(x2, pho_w[...], preferred_element_type=jnp.float32) + pho_b[...]


def _post3_body(accm, accs, sp, dis, ytab3m, ytab3s, phor,
                m_ew, m_new, m_b, s_ew, s_new, s_b,
                bnm_g, bnm_b, bns_g, bns_b, mean_o, logstd_o):
    s_ = sp[0] + sp[1]
    vm = jnp.dot(m_ew[...], m_new[64:], preferred_element_type=jnp.float32)
    vs = jnp.dot(s_ew[...], s_new[64:], preferred_element_type=jnp.float32)
    d = dis[...][:, None]
    mean_pre = d * (accm[0] + accm[1] + s_[:, None] * vm + ytab3m[...]) + m_b[...]
    mean_o[...] = _bn(mean_pre, bnm_g[...], bnm_b[...]) + phor[...]
    ls_pre = d * (accs[0] + accs[1] + s_[:, None] * vs + ytab3s[...]) + s_b[...]
    ls = _bn(ls_pre, bns_g[...], bns_b[...])
    logstd_o[...] = ls + _bn(ls, bns_g[...], bns_b[...])


def _tc(body, out_shape, *args):
    return pl.pallas_call(body, out_shape=out_shape)(*args)


def kernel(x, edge_index, edge_attr, params):
    p = params
    f32 = jnp.float32
    row = edge_index[0].reshape(NW, EPW)
    col = edge_index[1].reshape(NW, EPW)
    ea = edge_attr[:, 0].reshape(NW, EPW)
    pad = EPWP - EPW
    rowp = jnp.pad(row, ((0, 0), (0, pad)))
    colp = jnp.pad(col, ((0, 0), (0, pad)), constant_values=N)
    eap = jnp.pad(ea, ((0, 0), (0, pad)))
    row2 = rowp.reshape(NW, NCH, C)
    col2 = colp.reshape(NW, NCH, C)
    z1 = jnp.zeros((NPAD,), f32)
    z2 = jnp.zeros((C, F), f32)

    def _pass(yt):
        return _pass_kernel(yt, row2, col2, z2)[:, :N]

    degp = _deg_kernel(colp, z1)
    dis_pad = _tc(_dis_body, jax.ShapeDtypeStruct((NPAD,), f32), degp)
    dis = dis_pad[:N]
    sp = _s_kernel(rowp, colp, eap, dis_pad, z1)
    spn = sp[:, :N]

    ytab1 = _tc(_prep_body, jax.ShapeDtypeStruct((N, F), f32),
                x, dis, p["c1_nw"], p["c1_new"])
    acc1 = _pass(ytab1)

    ytab2, pih = _tc(
        _post1_body,
        [jax.ShapeDtypeStruct((N, F), f32), jax.ShapeDtypeStruct((N, F), f32)],
        acc1, spn, dis, ytab1, x, p["c1_ew"], p["c1_new"], p["c1_b"],
        p["bn1_g"], p["bn1_b"], p["l1_W"], p["l1_b"], p["pih_W"], p["pih_b"],
        p["c2_nw"], p["c2_new"])
    acc2 = _pass(ytab2)

    ytab3m, ytab3s, pho = _tc(
        _post2_body,
        [jax.ShapeDtypeStruct((N, 64), f32), jax.ShapeDtypeStruct((N, 64), f32),
         jax.ShapeDtypeStruct((N, 64), f32)],
        acc2, spn, dis, ytab2, pih, p["c2_ew"], p["c2_new"], p["c2_b"],
        p["bn2_g"], p["bn2_b"], p["l2_W"], p["l2_b"], p["pho_W"], p["pho_b"],
        p["c3m_nw"], p["c3m_new"], p["c3s_nw"], p["c3s_new"])
    acc3 = _pass(jnp.concatenate([ytab3m, ytab3s], axis=1))
    accm, accs = acc3[:, :, :64], acc3[:, :, 64:]

    mean, logstd = _tc(
        _post3_body,
        [jax.ShapeDtypeStruct((N, 64), f32), jax.ShapeDtypeStruct((N, 64), f32)],
        accm, accs, spn, dis, ytab3m, ytab3s, pho,
        p["c3m_ew"], p["c3m_new"], p["c3m_b"], p["c3s_ew"], p["c3s_new"], p["c3s_b"],
        p["bnm_g"], p["bnm_b"], p["bns_g"], p["bns_b"])
    return mean, logstd


# speedup vs baseline: 8.5181x; 1.2693x over previous
import functools

import jax
import jax.numpy as jnp
from jax import lax
from jax.experimental import pallas as pl
from jax.experimental.pallas import tpu as pltpu
from jax.experimental.pallas import tpu_sc as plsc

N = 10000
E = 320000
F = 128
NW = 32
EPW = E // NW
C = 128
NCH = 80
EPWP = NCH * C
NPAD = 10240
RPT = NPAD // 16

_MESH = plsc.VectorSubcoreMesh(core_axis_name="c", subcore_axis_name="s")
_SC_PARAMS = pltpu.CompilerParams(needs_layout_passes=False)

NH = 2048
RPH = NH // 16


def _reduce_hists(hist, hist_sh, blk, red, out, c, s):
    for h in range(NPAD // NH):
        pltpu.sync_copy(hist.at[pl.ds(h * NH, NH)], hist_sh.at[s])
        plsc.subcore_barrier()
        for r in range(16):
            pltpu.sync_copy(hist_sh.at[r, pl.ds(s * RPH, RPH)], blk.at[r])
        for v in range(RPH // 16):
            a = blk[0, pl.ds(v * 16, 16)]
            for r in range(1, 16):
                a = a + blk[r, pl.ds(v * 16, 16)]
            red[pl.ds(v * 16, 16)] = a
        pltpu.sync_copy(red, out.at[c, pl.ds(h * NH + s * RPH, RPH)])
        plsc.subcore_barrier()


@functools.partial(
    pl.kernel,
    out_type=jax.ShapeDtypeStruct((2, NPAD), jnp.float32),
    mesh=_MESH,
    compiler_params=_SC_PARAMS,
    scratch_types=[
        pltpu.VMEM((EPWP,), jnp.int32),
        pltpu.VMEM((NPAD,), jnp.float32),
        pltpu.VMEM((16, RPH), jnp.float32),
        pltpu.VMEM((RPH,), jnp.float32),
        pltpu.VMEM_SHARED((16, NH), jnp.float32),
    ],
)
def _deg_kernel(coli, zeros1d, out, colv, hist, blk, red, hist_sh):
    c = lax.axis_index("c")
    s = lax.axis_index("s")
    wid = s * 2 + c
    pltpu.sync_copy(coli.at[wid], colv)
    pltpu.sync_copy(zeros1d, hist)
    ones = jnp.ones((16,), jnp.float32)

    def body(i, carry):
        idx = colv[pl.ds(i * 16, 16)]
        plsc.addupdate_scatter(hist, [idx], ones)
        return carry

    lax.fori_loop(0, EPWP // 16, body, 0)
    _reduce_hists(hist, hist_sh, blk, red, out, c, s)


@functools.partial(
    pl.kernel,
    out_type=jax.ShapeDtypeStruct((2, NPAD), jnp.float32),
    mesh=_MESH,
    compiler_params=_SC_PARAMS,
    scratch_types=[
        pltpu.VMEM((EPWP,), jnp.int32),
        pltpu.VMEM((EPWP,), jnp.int32),
        pltpu.VMEM((EPWP,), jnp.float32),
        pltpu.VMEM((NPAD,), jnp.float32),
        pltpu.VMEM((NPAD,), jnp.float32),
        pltpu.VMEM((16, RPH), jnp.float32),
        pltpu.VMEM((RPH,), jnp.float32),
        pltpu.VMEM_SHARED((16, NH), jnp.float32),
    ],
)
def _s_kernel(rowi, coli, eai, dis_pad, zeros1d, out,
              rowv, colv, eav, disv, hist, blk, red, hist_sh):
    c = lax.axis_index("c")
    s = lax.axis_index("s")
    wid = s * 2 + c
    pltpu.sync_copy(rowi.at[wid], rowv)
    pltpu.sync_copy(coli.at[wid], colv)
    pltpu.sync_copy(eai.at[wid], eav)
    pltpu.sync_copy(dis_pad, disv)
    pltpu.sync_copy(zeros1d, hist)

    def body(i, carry):
        r16 = rowv[pl.ds(i * 16, 16)]
        c16 = colv[pl.ds(i * 16, 16)]
        e16 = eav[pl.ds(i * 16, 16)]
        dr = plsc.load_gather(disv, [r16])
        plsc.addupdate_scatter(hist, [c16], dr * e16)
        return carry

    lax.fori_loop(0, EPWP // 16, body, 0)
    _reduce_hists(hist, hist_sh, blk, red, out, c, s)


@functools.partial(
    pl.kernel,
    out_type=jax.ShapeDtypeStruct((2, NPAD, F), jnp.float32),
    mesh=_MESH,
    compiler_params=_SC_PARAMS,
    scratch_types=[
        pltpu.VMEM((NCH, C), jnp.int32),
        pltpu.VMEM((NCH, C), jnp.int32),
        pltpu.VMEM((C, F), jnp.float32),
        pltpu.VMEM((C, F), jnp.float32),
        pltpu.VMEM_SHARED((NPAD, F), jnp.float32),
        pltpu.SemaphoreType.DMA,
        pltpu.SemaphoreType.DMA,
        pltpu.SemaphoreType.DMA,
        pltpu.SemaphoreType.DMA,
    ],
)
def _pass_kernel(ytab, rowi, coli, zeros2d, out,
                 rowv, colv, bufa, bufb, acc, ga, gb, sa, sb):
    c = lax.axis_index("c")
    s = lax.axis_index("s")
    wid = s * 2 + c
    pltpu.sync_copy(rowi.at[wid], rowv)
    pltpu.sync_copy(coli.at[wid], colv)
    pltpu.sync_copy(zeros2d, bufa)
    for r0 in range(0, RPT, C):
        pltpu.sync_copy(bufa, acc.at[pl.ds(s * RPT + r0, C)])
    plsc.subcore_barrier()

    def body(j, carry):
        pltpu.async_copy(ytab.at[rowv.at[j]], bufa, ga).wait()
        pltpu.async_copy(bufa, acc.at[colv.at[j]], sa, add=True).wait()
        return carry

    lax.fori_loop(0, NCH, body, 0)
    plsc.subcore_barrier()
    for r0 in range(0, RPT, C):
        pltpu.sync_copy(acc.at[pl.ds(s * RPT + r0, C)], bufa)
        pltpu.sync_copy(bufa, out.at[c, pl.ds(s * RPT + r0, C)])


def _bn(z, g, b):
    mu = jnp.mean(z, axis=0)
    var = jnp.mean((z - mu) ** 2, axis=0)
    return g * (z - mu) * lax.rsqrt(var + 1e-5) + b


def _dis_body(degp, dis_pad):
    deg = degp[0] + degp[1] + (1.0 + 1e-6)
    dis_pad[...] = lax.rsqrt(deg)


def _prep_body(x, dis, nw, new_w, ytab):
    w = jnp.dot(nw[...], new_w[:F], preferred_element_type=jnp.float32)
    y = jnp.dot(x[...], w, preferred_element_type=jnp.float32)
    ytab[...] = dis[...][:, None] * y


def _post1_body(accp, sp, dis, ytab1, x, ew, new_w, b1, bn_g, bn_b,
                l1_w, l1_b, pih_w, pih_b, c2_nw, c2_new, ytab2, pih):
    s_ = sp[0] + sp[1]
    v1 = jnp.dot(ew[...], new_w[F:], preferred_element_type=jnp.float32)
    agg = accp[0] + accp[1] + s_[:, None] * v1 + ytab1[...]
    out1 = dis[...][:, None] * agg + b1[...]
    x1 = jax.nn.relu(_bn(out1, bn_g[...], bn_b[...])) + x[...]
    z2 = jnp.dot(x1, l1_w[...], preferred_element_type=jnp.float32) + l1_b[...]
    w2 = jnp.dot(c2_nw[...], c2_new[:F], preferred_element_type=jnp.float32)
    ytab2[...] = dis[...][:, None] * jnp.dot(z2, w2, preferred_element_type=jnp.float32)
    pih[...] = jnp.dot(x1, pih_w[...], preferred_element_type=jnp.float32) + pih_b[...]


def _post2_body(accp, sp, dis, ytab2, pihr, ew, new_w, b2, bn_g, bn_b,
                l2_w, l2_b, pho_w, pho_b, m_nw, m_new, s_nw, s_new,
                ytab3m, ytab3s, pho):
    s_ = sp[0] + sp[1]
    v2 = jnp.dot(ew[...], new_w[F:], preferred_element_type=jnp.float32)
    agg = accp[0] + accp[1] + s_[:, None] * v2 + ytab2[...]
    out2 = dis[...][:, None] * agg + b2[...]
    x2 = jax.nn.relu(_bn(out2, bn_g[...], bn_b[...])) + pihr[...]
    x3 = jnp.dot(x2, l2_w[...], preferred_element_type=jnp.float32) + l2_b[...]
    wm = jnp.dot(m_nw[...], m_new[:64], preferred_element_type=jnp.float32)
    ws = jnp.dot(s_nw[...], s_new[:64], preferred_element_type=jnp.float32)
    ytab3m[...] = dis[...][:, None] * jnp.dot(x3, wm, preferred_element_type=jnp.float32)
    ytab3s[...] = dis[...][:, None] * jnp.dot(x3, ws, preferred_element_type=jnp.float32)
    pho[...] = jnp.dot(x2, pho_w[...], preferred_element_type=jnp.float32) + pho_b[...]


def _post3_body(accm, accs, sp, dis, ytab3m, ytab3s, phor,
                m_ew, m_new, m_b, s_ew, s_new, s_b,
                bnm_g, bnm_b, bns_g, bns_b, mean_o, logstd_o):
    s_ = sp[0] + sp[1]
    vm = jnp.dot(m_ew[...], m_new[64:], preferred_element_type=jnp.float32)
    vs = jnp.dot(s_ew[...], s_new[64:], preferred_element_type=jnp.float32)
    d = dis[...][:, None]
    mean_pre = d * (accm[0] + accm[1] + s_[:, None] * vm + ytab3m[...]) + m_b[...]
    mean_o[...] = _bn(mean_pre, bnm_g[...], bnm_b[...]) + phor[...]
    ls_pre = d * (accs[0] + accs[1] + s_[:, None] * vs + ytab3s[...]) + s_b[...]
    ls = _bn(ls_pre, bns_g[...], bns_b[...])
    logstd_o[...] = ls + _bn(ls, bns_g[...], bns_b[...])


def _tc(body, out_shape, *args):
    return pl.pallas_call(body, out_shape=out_shape)(*args)


def kernel(x, edge_index, edge_attr, params):
    p = params
    f32 = jnp.float32
    row = edge_index[0].reshape(NW, EPW)
    col = edge_index[1].reshape(NW, EPW)
    ea = edge_attr[:, 0].reshape(NW, EPW)
    pad = EPWP - EPW
    rowp = jnp.pad(row, ((0, 0), (0, pad)))
    colp = jnp.pad(col, ((0, 0), (0, pad)), constant_values=N)
    eap = jnp.pad(ea, ((0, 0), (0, pad)))
    row2 = rowp.reshape(NW, NCH, C)
    col2 = colp.reshape(NW, NCH, C)
    z1 = jnp.zeros((NPAD,), f32)
    z2 = jnp.zeros((C, F), f32)

    def _pass(yt):
        return _pass_kernel(yt, row2, col2, z2)[:, :N]

    degp = _deg_kernel(colp, z1)
    dis_pad = _tc(_dis_body, jax.ShapeDtypeStruct((NPAD,), f32), degp)
    dis = dis_pad[:N]
    sp = _s_kernel(rowp, colp, eap, dis_pad, z1)
    spn = sp[:, :N]

    ytab1 = _tc(_prep_body, jax.ShapeDtypeStruct((N, F), f32),
                x, dis, p["c1_nw"], p["c1_new"])
    acc1 = _pass(ytab1)

    ytab2, pih = _tc(
        _post1_body,
        [jax.ShapeDtypeStruct((N, F), f32), jax.ShapeDtypeStruct((N, F), f32)],
        acc1, spn, dis, ytab1, x, p["c1_ew"], p["c1_new"], p["c1_b"],
        p["bn1_g"], p["bn1_b"], p["l1_W"], p["l1_b"], p["pih_W"], p["pih_b"],
        p["c2_nw"], p["c2_new"])
    acc2 = _pass(ytab2)

    ytab3m, ytab3s, pho = _tc(
        _post2_body,
        [jax.ShapeDtypeStruct((N, 64), f32), jax.ShapeDtypeStruct((N, 64), f32),
         jax.ShapeDtypeStruct((N, 64), f32)],
        acc2, spn, dis, ytab2, pih, p["c2_ew"], p["c2_new"], p["c2_b"],
        p["bn2_g"], p["bn2_b"], p["l2_W"], p["l2_b"], p["pho_W"], p["pho_b"],
        p["c3m_nw"], p["c3m_new"], p["c3s_nw"], p["c3s_new"])
    acc3 = _pass(jnp.concatenate([ytab3m, ytab3s], axis=1))
    accm, accs = acc3[:, :, :64], acc3[:, :, 64:]

    mean, logstd = _tc(
        _post3_body,
        [jax.ShapeDtypeStruct((N, 64), f32), jax.ShapeDtypeStruct((N, 64), f32)],
        accm, accs, spn, dis, ytab3m, ytab3s, pho,
        p["c3m_ew"], p["c3m_new"], p["c3m_b"], p["c3s_ew"], p["c3s_new"], p["c3s_b"],
        p["bnm_g"], p["bnm_b"], p["bns_g"], p["bns_b"])
    return mean, logstd
